# Initial kernel scaffold; baseline (speedup 1.0000x reference)
import jax
import jax.numpy as jnp
from jax.experimental import pallas as pl


def kernel(node_input, node_attr, edge_src, edge_dst, edge_attr, edge_scalars, W_sc, W_lin1, W_fc1, W_fc2, W_lin2, W_lin3):
    raise NotImplementedError("write your pallas kernel here")


# speedup vs baseline: 2.2589x; 2.2589x over previous
import functools

import jax
import jax.numpy as jnp
import numpy as np
from jax import lax
from jax.experimental import pallas as pl
from jax.experimental.pallas import tpu as pltpu
from jax.experimental.pallas import tpu_sc as plsc


def _node_pre_body(x_ref, a_ref, w_ref, o_ref):
    x = x_ref[...]
    o_ref[...] = jnp.dot(x, w_ref[...], preferred_element_type=jnp.float32) * a_ref[...]


def _node_pre(x, attr, w, block):
    n, d = x.shape
    do = w.shape[1]
    return pl.pallas_call(
        _node_pre_body,
        grid=(n // block,),
        in_specs=[
            pl.BlockSpec((block, d), lambda i: (i, 0)),
            pl.BlockSpec((block, 1), lambda i: (i, 0)),
            pl.BlockSpec((d, do), lambda i: (0, 0)),
        ],
        out_specs=pl.BlockSpec((block, do), lambda i: (i, 0)),
        out_shape=jax.ShapeDtypeStruct((n, do), jnp.float32),
    )(x, attr, w)


def _edge_w_body(s_ref, a_ref, w1_ref, w2_ref, o_ref):
    h = jnp.dot(s_ref[...], w1_ref[...], preferred_element_type=jnp.float32)
    h = jax.nn.silu(h)
    w = jnp.dot(h, w2_ref[...], preferred_element_type=jnp.float32)
    o_ref[...] = w * a_ref[...]


def _edge_w(scalars, attr, w1, w2, block):
    e, f0 = scalars.shape
    f1, d = w2.shape
    return pl.pallas_call(
        _edge_w_body,
        grid=(e // block,),
        in_specs=[
            pl.BlockSpec((block, f0), lambda i: (i, 0)),
            pl.BlockSpec((block, 1), lambda i: (i, 0)),
            pl.BlockSpec((f0, f1), lambda i: (0, 0)),
            pl.BlockSpec((f1, d), lambda i: (0, 0)),
        ],
        out_specs=pl.BlockSpec((block, d), lambda i: (i, 0)),
        out_shape=jax.ShapeDtypeStruct((e, d), jnp.float32),
    )(scalars, attr, w1, w2)


def _make_sc_scatter(n, e, d, chunk):
    nc, ns = 2, 16
    nw = nc * ns
    e_per_w = e // nw
    n_chunks = e_per_w // chunk
    zrows = 16
    n_zchunks = pl.cdiv(n, zrows)
    zper_tile = pl.cdiv(n_zchunks, ns)
    mesh = plsc.VectorSubcoreMesh(core_axis_name="c", subcore_axis_name="s",
                                  num_cores=nc, num_subcores=ns)

    @functools.partial(
        pl.kernel,
        mesh=mesh,
        out_type=jax.ShapeDtypeStruct((nc, n, d), jnp.float32),
        scratch_types=[
            pltpu.VMEM((chunk,), jnp.int32),
            pltpu.VMEM((1, chunk), jnp.int32),
            pltpu.VMEM((chunk, d), jnp.float32),
            pltpu.VMEM((chunk, d), jnp.float32),
            pltpu.VMEM((zrows, d), jnp.float32),
            pltpu.VMEM_SHARED((n, d), jnp.float32),
            pltpu.SemaphoreType.DMA,
            pltpu.SemaphoreType.DMA,
        ],
    )
    def sc_kernel(src_hbm, dst_hbm, w_hbm, nf_hbm, out_hbm,
                  sidx_v, didx_v, rows_v, wrow_v, z_v, acc, sem_g, sem_w):
        cid = lax.axis_index("c")
        sid = lax.axis_index("s")
        wid = sid * nc + cid

        zv = jnp.zeros((16,), jnp.float32)

        def zbody(i, _):
            for j in range(d // 16):
                z_v[i, pl.ds(j * 16, 16)] = zv
            return 0

        lax.fori_loop(0, zrows, zbody, 0)

        def zcopy(i, _):
            c = sid + i * ns
            @pl.when(c < n_zchunks)
            def _():
                pltpu.sync_copy(z_v, acc.at[pl.ds(c * zrows, zrows)])
            return 0

        lax.fori_loop(0, zper_tile, zcopy, 0)
        plsc.subcore_barrier()

        def body(k, _):
            base = wid * e_per_w + k * chunk
            pltpu.sync_copy(src_hbm.at[pl.ds(base, chunk)], sidx_v)
            pltpu.sync_copy(dst_hbm.at[pl.ds(base, chunk)], didx_v.at[0])
            cp_w = pltpu.async_copy(w_hbm.at[pl.ds(base, chunk)], wrow_v, sem_w)
            cp_g = pltpu.async_copy(nf_hbm.at[sidx_v], rows_v, sem_g)
            cp_g.wait()
            cp_w.wait()

            def mul_body(t, _):
                for j in range(d // 16):
                    sl = pl.ds(j * 16, 16)
                    rows_v[t, sl] = rows_v[t, sl] * wrow_v[t, sl]
                return 0

            lax.fori_loop(0, chunk, mul_body, 0)
            pltpu.sync_copy(rows_v, acc.at[didx_v.at[0]], add=True)
            return 0

        lax.fori_loop(0, n_chunks, body, 0)

        plsc.subcore_barrier()

        def dcopy(i, _):
            c = sid + i * ns
            @pl.when(c < n_zchunks)
            def _():
                pltpu.sync_copy(acc.at[pl.ds(c * zrows, zrows)],
                                out_hbm.at[cid, pl.ds(c * zrows, zrows)])
            return 0

        lax.fori_loop(0, zper_tile, dcopy, 0)

    return sc_kernel


def _post_body(p0_ref, p1_ref, s_ref, a_ref, w2_ref, w3_ref, o_ref):
    h = p0_ref[...] + p1_ref[...]
    a = a_ref[...]
    conv = jnp.dot(h, w2_ref[...], preferred_element_type=jnp.float32) * a
    ang = jnp.sum(h * w3_ref[...], axis=1, keepdims=True) * a
    o_ref[...] = jnp.cos(ang) * s_ref[...] + jnp.sin(ang) * conv


def _post(p0, p1, node_self, attr, w2, w3, block):
    n, d = p0.shape
    return pl.pallas_call(
        _post_body,
        grid=(n // block,),
        in_specs=[
            pl.BlockSpec((block, d), lambda i: (i, 0)),
            pl.BlockSpec((block, d), lambda i: (i, 0)),
            pl.BlockSpec((block, d), lambda i: (i, 0)),
            pl.BlockSpec((block, 1), lambda i: (i, 0)),
            pl.BlockSpec((d, d), lambda i: (0, 0)),
            pl.BlockSpec((1, d), lambda i: (0, 0)),
        ],
        out_specs=pl.BlockSpec((block, d), lambda i: (i, 0)),
        out_shape=jax.ShapeDtypeStruct((n, d), jnp.float32),
    )(p0, p1, node_self, attr, w2, w3)


NUM_NEIGHBORS = 32.0


def kernel(node_input, node_attr, edge_src, edge_dst, edge_attr, edge_scalars,
           W_sc, W_lin1, W_fc1, W_fc2, W_lin2, W_lin3):
    n, d = node_input.shape
    e = edge_src.shape[0]
    f0, f1 = W_fc1.shape

    wn = jnp.concatenate([W_sc[:, 0, :], W_lin1[:, 0, :]], axis=1) / np.sqrt(d)
    w1 = W_fc1 / np.sqrt(f0)
    w2 = W_fc2 / (np.sqrt(f1) * np.sqrt(NUM_NEIGHBORS))
    wl2 = W_lin2[:, 0, :] / np.sqrt(d)
    wl3 = (0.1 / np.sqrt(d)) * W_lin3[:, 0, :].reshape(1, d)

    pre = _node_pre(node_input, node_attr, wn, block=1000)
    node_self = pre[:, :d]
    node_feat = pre[:, d:]

    ew = _edge_w(edge_scalars, edge_attr, w1, w2, block=3200)

    sc = _make_sc_scatter(n, e, d, chunk=80)
    partial = sc(edge_src.astype(jnp.int32), edge_dst.astype(jnp.int32),
                 ew, node_feat)

    return _post(partial[0], partial[1], node_self, node_attr,
                 wl2, wl3, block=1000)
